# baseline (speedup 1.0000x reference)
import functools

import jax
import jax.numpy as jnp
from jax import lax
from jax.experimental import pallas as pl
from jax.experimental.pallas import tpu as pltpu
from jax.experimental.pallas import tpu_sc as plsc

NUM_CORES = 2
NUM_SUBCORES = 16
NUM_WORKERS = NUM_CORES * NUM_SUBCORES

CHUNK = 800
NBUF = 4


@functools.partial(jax.jit, static_argnames=("n_idx", "dim"))
def _sc_gather(x_flat, table, n_idx, dim):
    per_w = n_idx // NUM_WORKERS
    n_chunks = per_w // CHUNK
    n_outer = n_chunks // NBUF
    mesh = plsc.VectorSubcoreMesh(core_axis_name="c", subcore_axis_name="s")

    @functools.partial(
        pl.kernel,
        mesh=mesh,
        out_type=jax.ShapeDtypeStruct((n_idx, dim), jnp.float32),
        scratch_types=[
            pltpu.VMEM((per_w,), jnp.int32),
            [pltpu.VMEM((CHUNK, dim), jnp.float32) for _ in range(NBUF)],
            [pltpu.SemaphoreType.DMA for _ in range(NBUF)],
            [pltpu.SemaphoreType.DMA for _ in range(NBUF)],
        ],
        compiler_params=pltpu.CompilerParams(use_tc_tiling_on_sc=False),
    )
    def k(x_hbm, table_hbm, out_hbm, idx_v, rows, gsem, wsem):
        wid = lax.axis_index("s") * NUM_CORES + lax.axis_index("c")
        base = pl.multiple_of(wid * per_w, per_w)
        pltpu.sync_copy(x_hbm.at[pl.ds(base, per_w)], idx_v)

        def gather(chunk_i, b):
            off = pl.multiple_of(chunk_i * CHUNK, CHUNK)
            pltpu.async_copy(
                table_hbm.at[idx_v.at[pl.ds(off, CHUNK)]], rows[b], gsem[b]
            )

        def writeback(chunk_i, b):
            off = pl.multiple_of(base + chunk_i * CHUNK, CHUNK)
            pltpu.async_copy(rows[b], out_hbm.at[pl.ds(off, CHUNK)], wsem[b])

        for b in range(NBUF):
            gather(b, b)

        def body(g, _):
            i0 = g * NBUF
            for b in range(NBUF):
                i = i0 + b
                pltpu.make_async_copy(
                    table_hbm.at[idx_v.at[pl.ds(0, CHUNK)]], rows[b], gsem[b]
                ).wait()
                writeback(i, b)
                @pl.when(i + NBUF < n_chunks)
                def _():
                    pltpu.make_async_copy(
                        rows[b], out_hbm.at[pl.ds(base, CHUNK)], wsem[b]
                    ).wait()
                    gather(i + NBUF, b)
            return 0

        lax.fori_loop(0, n_outer, body, 0)

        for b in range(NBUF):
            pltpu.make_async_copy(
                rows[b], out_hbm.at[pl.ds(base, CHUNK)], wsem[b]
            ).wait()

    return k(x_flat, table)


def kernel(x, table):
    b, l = x.shape
    dim = table.shape[1]
    x_flat = x.reshape(b * l).astype(jnp.int32)
    out = _sc_gather(x_flat, table, b * l, dim)
    return out.reshape(b, l, dim)


# speedup vs baseline: 1.3107x; 1.3107x over previous
import functools

import jax
import jax.numpy as jnp
from jax import lax
from jax.experimental import pallas as pl
from jax.experimental.pallas import tpu as pltpu
from jax.experimental.pallas import tpu_sc as plsc

NUM_CORES = 2
NUM_SUBCORES = 16
NUM_WORKERS = NUM_CORES * NUM_SUBCORES

BLK = 512
NBUF = 2
LANES = 16


@functools.partial(jax.jit, static_argnames=("batch", "hist", "dim"))
def _sc_gather(xt, table, batch, hist, dim):
    blocks = batch // BLK
    n_units = hist * blocks
    per_w = n_units // NUM_WORKERS
    n_outer = per_w // NBUF
    mesh = plsc.VectorSubcoreMesh(core_axis_name="c", subcore_axis_name="s")

    @functools.partial(
        pl.kernel,
        mesh=mesh,
        out_type=jax.ShapeDtypeStruct((hist, dim, batch), jnp.float32),
        scratch_types=[
            [pltpu.VMEM((1, BLK), jnp.int32) for _ in range(NBUF)],
            [pltpu.VMEM((BLK, dim), jnp.float32) for _ in range(NBUF)],
            pltpu.VMEM((dim, BLK), jnp.float32),
            [pltpu.SemaphoreType.DMA for _ in range(NBUF)],
            pltpu.SemaphoreType.DMA,
        ],
        compiler_params=pltpu.CompilerParams(
            use_tc_tiling_on_sc=False, needs_layout_passes=False
        ),
    )
    def k(xt_hbm, table_hbm, out_hbm, idxs, rows, tr, gsem, wsem):
        wid = lax.axis_index("s") * NUM_CORES + lax.axis_index("c")
        u_base = wid * per_w
        lane_iota = lax.iota(jnp.int32, LANES)

        def start_gather(u_local, b):
            u = u_base + u_local
            l = u // blocks
            b0 = pl.multiple_of((u % blocks) * BLK, BLK)
            pltpu.sync_copy(xt_hbm.at[pl.ds(l, 1), pl.ds(b0, BLK)], idxs[b])
            pltpu.async_copy(table_hbm.at[idxs[b].at[0]], rows[b], gsem[b])

        def transpose_block(b):
            def body_j(j, _):
                row_ids = j * LANES + lane_iota
                for d in range(dim):
                    col_ids = jnp.full((LANES,), d, jnp.int32)
                    v = plsc.load_gather(rows[b], [row_ids, col_ids])
                    tr[d, pl.ds(j * LANES, LANES)] = v
                return 0

            lax.fori_loop(0, BLK // LANES, body_j, 0)

        def start_writeback(u_local):
            u = u_base + u_local
            l = u // blocks
            b0 = pl.multiple_of((u % blocks) * BLK, BLK)
            pltpu.async_copy(tr, out_hbm.at[l, :, pl.ds(b0, BLK)], wsem)

        def wait_gather(b):
            pltpu.make_async_copy(
                table_hbm.at[idxs[b].at[0]], rows[b], gsem[b]
            ).wait()

        def wait_writeback():
            pltpu.make_async_copy(tr, out_hbm.at[0, :, pl.ds(0, BLK)], wsem).wait()

        for b in range(NBUF):
            start_gather(b, b)

        def body(g, _):
            u0 = g * NBUF
            for b in range(NBUF):
                u_local = u0 + b
                wait_gather(b)

                @pl.when(u_local > 0)
                def _():
                    wait_writeback()

                transpose_block(b)
                start_writeback(u_local)

                @pl.when(u_local + NBUF < per_w)
                def _():
                    start_gather(u_local + NBUF, b)
            return 0

        lax.fori_loop(0, n_outer, body, 0)
        wait_writeback()

    return k(xt, table)


def kernel(x, table):
    b, l = x.shape
    dim = table.shape[1]
    xt = jnp.transpose(x).astype(jnp.int32)
    out_t = _sc_gather(xt, table, b, l, dim)
    return jnp.transpose(out_t, (2, 0, 1))


# speedup vs baseline: 1.3161x; 1.0041x over previous
import functools

import jax
import jax.numpy as jnp
from jax import lax
from jax.experimental import pallas as pl
from jax.experimental.pallas import tpu as pltpu
from jax.experimental.pallas import tpu_sc as plsc

NUM_CORES = 2
NUM_SUBCORES = 16
NUM_WORKERS = NUM_CORES * NUM_SUBCORES

BLK = 512
NBUF = 2
LANES = 16


@functools.partial(jax.jit, static_argnames=("batch", "hist", "dim"))
def _sc_gather(xt, table, batch, hist, dim):
    blocks = batch // BLK
    n_units = hist * blocks
    per_w = n_units // NUM_WORKERS
    n_outer = per_w // NBUF
    mesh = plsc.VectorSubcoreMesh(core_axis_name="c", subcore_axis_name="s")

    @functools.partial(
        pl.kernel,
        mesh=mesh,
        out_type=jax.ShapeDtypeStruct((hist, dim, batch), jnp.float32),
        scratch_types=[
            [pltpu.VMEM((1, BLK), jnp.int32) for _ in range(NBUF)],
            [pltpu.VMEM((BLK, dim), jnp.float32) for _ in range(NBUF)],
            [pltpu.VMEM((dim, BLK), jnp.float32) for _ in range(NBUF)],
            [pltpu.SemaphoreType.DMA for _ in range(NBUF)],
            [pltpu.SemaphoreType.DMA for _ in range(NBUF)],
        ],
        compiler_params=pltpu.CompilerParams(
            use_tc_tiling_on_sc=False, needs_layout_passes=False
        ),
    )
    def k(xt_hbm, table_hbm, out_hbm, idxs, rows, tr, gsem, wsem):
        wid = lax.axis_index("s") * NUM_CORES + lax.axis_index("c")
        u_base = wid * per_w
        lane_iota = lax.iota(jnp.int32, LANES)

        def start_gather(u_local, b):
            u = u_base + u_local
            l = u // blocks
            b0 = pl.multiple_of((u % blocks) * BLK, BLK)
            pltpu.sync_copy(xt_hbm.at[pl.ds(l, 1), pl.ds(b0, BLK)], idxs[b])
            pltpu.async_copy(table_hbm.at[idxs[b].at[0]], rows[b], gsem[b])

        def transpose_block(b):
            def body_j(j, _):
                row_ids = j * LANES + lane_iota
                for d in range(dim):
                    col_ids = jnp.full((LANES,), d, jnp.int32)
                    v = plsc.load_gather(rows[b], [row_ids, col_ids])
                    tr[b][d, pl.ds(j * LANES, LANES)] = v
                return 0

            lax.fori_loop(0, BLK // LANES, body_j, 0)

        def start_writeback(u_local, b):
            u = u_base + u_local
            l = u // blocks
            b0 = pl.multiple_of((u % blocks) * BLK, BLK)
            pltpu.async_copy(tr[b], out_hbm.at[l, :, pl.ds(b0, BLK)], wsem[b])

        def wait_gather(b):
            pltpu.make_async_copy(
                table_hbm.at[idxs[b].at[0]], rows[b], gsem[b]
            ).wait()

        def wait_writeback(b):
            pltpu.make_async_copy(
                tr[b], out_hbm.at[0, :, pl.ds(0, BLK)], wsem[b]
            ).wait()

        for b in range(NBUF):
            start_gather(b, b)

        def body(g, _):
            u0 = g * NBUF
            for b in range(NBUF):
                u_local = u0 + b
                wait_gather(b)

                @pl.when(u_local >= NBUF)
                def _():
                    wait_writeback(b)

                transpose_block(b)
                start_writeback(u_local, b)

                @pl.when(u_local + NBUF < per_w)
                def _():
                    start_gather(u_local + NBUF, b)
            return 0

        lax.fori_loop(0, n_outer, body, 0)
        for b in range(NBUF):
            wait_writeback(b)

    return k(xt, table)


def kernel(x, table):
    b, l = x.shape
    dim = table.shape[1]
    xt = jnp.transpose(x).astype(jnp.int32)
    out_t = _sc_gather(xt, table, b, l, dim)
    return jnp.transpose(out_t, (2, 0, 1))


# speedup vs baseline: 1.3515x; 1.0269x over previous
import functools

import jax
import jax.numpy as jnp
from jax import lax
from jax.experimental import pallas as pl
from jax.experimental.pallas import tpu as pltpu
from jax.experimental.pallas import tpu_sc as plsc

NUM_CORES = 2
NUM_SUBCORES = 16
NUM_WORKERS = NUM_CORES * NUM_SUBCORES

BLK = 512
NBUF = 2
LANES = 16


@functools.partial(jax.jit, static_argnames=("batch", "hist", "dim"))
def _sc_gather(xt, table, batch, hist, dim):
    blocks = batch // BLK
    n_units = hist * blocks
    per_w = n_units // NUM_WORKERS
    n_outer = per_w // NBUF
    mesh = plsc.VectorSubcoreMesh(core_axis_name="c", subcore_axis_name="s")

    @functools.partial(
        pl.kernel,
        mesh=mesh,
        out_type=jax.ShapeDtypeStruct((hist, dim, batch), jnp.float32),
        scratch_types=[
            [pltpu.VMEM((1, BLK), jnp.int32) for _ in range(NBUF)],
            [pltpu.VMEM((BLK, dim), jnp.float32) for _ in range(NBUF)],
            [pltpu.VMEM((dim, BLK), jnp.float32) for _ in range(NBUF)],
            [pltpu.SemaphoreType.DMA for _ in range(NBUF)],
            [pltpu.SemaphoreType.DMA for _ in range(NBUF)],
            [pltpu.SemaphoreType.DMA for _ in range(NBUF)],
        ],
        compiler_params=pltpu.CompilerParams(
            use_tc_tiling_on_sc=False, needs_layout_passes=False
        ),
    )
    def k(xt_hbm, table_hbm, out_hbm, idxs, rows, tr, gsem, wsem, isem):
        wid = lax.axis_index("s") * NUM_CORES + lax.axis_index("c")
        u_base = wid * per_w
        lane_iota = lax.iota(jnp.int32, LANES)

        def start_idx(u_local, b):
            u = u_base + u_local
            l = u // blocks
            b0 = pl.multiple_of((u % blocks) * BLK, BLK)
            pltpu.async_copy(
                xt_hbm.at[pl.ds(l, 1), pl.ds(b0, BLK)], idxs[b], isem[b]
            )

        def start_gather(u_local, b):
            pltpu.make_async_copy(
                xt_hbm.at[pl.ds(0, 1), pl.ds(0, BLK)], idxs[b], isem[b]
            ).wait()
            pltpu.async_copy(table_hbm.at[idxs[b].at[0]], rows[b], gsem[b])

        def transpose_block(b):
            def body_j(j, _):
                row_ids = j * LANES + lane_iota
                for d in range(dim):
                    col_ids = jnp.full((LANES,), d, jnp.int32)
                    v = plsc.load_gather(rows[b], [row_ids, col_ids])
                    tr[b][d, pl.ds(j * LANES, LANES)] = v
                return 0

            lax.fori_loop(0, BLK // LANES, body_j, 0)

        def start_writeback(u_local, b):
            u = u_base + u_local
            l = u // blocks
            b0 = pl.multiple_of((u % blocks) * BLK, BLK)
            pltpu.async_copy(tr[b], out_hbm.at[l, :, pl.ds(b0, BLK)], wsem[b])

        def wait_gather(b):
            pltpu.make_async_copy(
                table_hbm.at[idxs[b].at[0]], rows[b], gsem[b]
            ).wait()

        def wait_writeback(b):
            pltpu.make_async_copy(
                tr[b], out_hbm.at[0, :, pl.ds(0, BLK)], wsem[b]
            ).wait()

        for b in range(NBUF):
            start_idx(b, b)
        for b in range(NBUF):
            start_gather(b, b)

        def body(g, _):
            u0 = g * NBUF
            for b in range(NBUF):
                u_local = u0 + b
                wait_gather(b)

                @pl.when(u_local + NBUF < per_w)
                def _():
                    start_idx(u_local + NBUF, b)

                @pl.when(u_local >= NBUF)
                def _():
                    wait_writeback(b)

                transpose_block(b)
                start_writeback(u_local, b)

                @pl.when(u_local + NBUF < per_w)
                def _():
                    start_gather(u_local + NBUF, b)
            return 0

        lax.fori_loop(0, n_outer, body, 0)
        for b in range(NBUF):
            wait_writeback(b)

    return k(xt, table)


def kernel(x, table):
    b, l = x.shape
    dim = table.shape[1]
    xt = jnp.transpose(x).astype(jnp.int32)
    out_t = _sc_gather(xt, table, b, l, dim)
    return jnp.transpose(out_t, (2, 0, 1))
